# Initial kernel scaffold; baseline (speedup 1.0000x reference)
import jax
import jax.numpy as jnp
from jax.experimental import pallas as pl


def kernel(x, pos, W1, b1, W2, b2):
    raise NotImplementedError("write your pallas kernel here")


# speedup vs baseline: 27.1456x; 27.1456x over previous
import functools

import jax
import jax.numpy as jnp
from jax import lax
from jax.experimental import pallas as pl
from jax.experimental.pallas import tpu as pltpu
from jax.experimental.pallas import tpu_sc as plsc

_K = 8
_BM = 2048
_BR = 256
_BR2 = 512
_NC = 2
_NS = 16
_NW = _NC * _NS
_CH = 512


def _mlp1_body(x_ref, w1_ref, b1_ref, a_ref, y_ref):
    xb = x_ref[...]
    w = w1_ref[...]
    f = xb.shape[1]
    a_ref[...] = (
        jnp.dot(xb, w[:f], preferred_element_type=jnp.float32) + b1_ref[...]
    )
    y_ref[...] = jnp.dot(xb, w[f:], preferred_element_type=jnp.float32)


def _knn_body(pos_ref, post_ref, idx_ref):
    p = pos_ref[0]
    pt = post_ref[0]
    n = pt.shape[1]
    g = jnp.dot(p, pt, preferred_element_type=jnp.float32)
    sqi = jnp.sum(p * p, axis=1, keepdims=True)
    sqj = jnp.sum(pt * pt, axis=0, keepdims=True)
    d = (sqi - 2.0 * g) + sqj
    cols = []
    vals = d
    col_iota = lax.broadcasted_iota(jnp.int32, d.shape, 1)
    for _ in range(_K):
        am = jnp.argmin(vals, axis=1).astype(jnp.int32)
        cols.append(am)
        vals = jnp.where(col_iota == am[:, None], jnp.inf, vals)
    idx = jnp.stack(cols, axis=1)
    idx_ref[...] = idx + pl.program_id(0) * n


def _mlp2_body(a_ref, yk_ref, w2_ref, b2_ref, o_ref):
    a = a_ref[...]
    br2, f = a.shape
    yk = yk_ref[...].reshape(br2, _K, f)
    h = jnp.maximum(a[:, None, :] + yk, 0.0)
    o = jnp.dot(
        h.reshape(br2 * _K, f), w2_ref[...], preferred_element_type=jnp.float32
    )
    o_ref[...] = jnp.max(o.reshape(br2, _K, f), axis=1) + b2_ref[...]


@functools.cache
def _make_gather(rows, d):
    per_w = rows // _NW
    n_ch = per_w // _CH
    mesh = plsc.VectorSubcoreMesh(core_axis_name="c", subcore_axis_name="s")

    @functools.partial(
        pl.kernel,
        out_type=jax.ShapeDtypeStruct((rows, d), jnp.float32),
        mesh=mesh,
        scratch_types=[
            pltpu.VMEM((_CH,), jnp.int32),
            pltpu.VMEM((_CH, d), jnp.float32),
            pltpu.SemaphoreType.DMA,
        ],
    )
    def gather(table_hbm, idx_hbm, out_hbm, idx_v, rows_v, sem):
        wid = lax.axis_index("s") * _NC + lax.axis_index("c")
        base = wid * per_w

        def body(i, carry):
            off = base + i * _CH
            pltpu.sync_copy(idx_hbm.at[pl.ds(off, _CH)], idx_v)
            pltpu.async_copy(table_hbm.at[idx_v], rows_v, sem).wait()
            pltpu.sync_copy(rows_v, out_hbm.at[pl.ds(off, _CH)])
            return carry

        lax.fori_loop(0, n_ch, body, 0)

    return gather


def kernel(x, pos, W1, b1, W2, b2):
    B, N, F = x.shape
    BN = B * N
    xf = x.reshape(BN, F)

    a, y = pl.pallas_call(
        _mlp1_body,
        grid=(BN // _BM,),
        in_specs=[
            pl.BlockSpec((_BM, F), lambda i: (i, 0)),
            pl.BlockSpec((2 * F, F), lambda i: (0, 0)),
            pl.BlockSpec((1, F), lambda i: (0, 0)),
        ],
        out_specs=[
            pl.BlockSpec((_BM, F), lambda i: (i, 0)),
            pl.BlockSpec((_BM, F), lambda i: (i, 0)),
        ],
        out_shape=[
            jax.ShapeDtypeStruct((BN, F), jnp.float32),
            jax.ShapeDtypeStruct((BN, F), jnp.float32),
        ],
    )(xf, W1, b1.reshape(1, F))

    pos_pad = jnp.pad(pos, ((0, 0), (0, 0), (0, 5)))
    post = jnp.transpose(pos_pad, (0, 2, 1))
    nb = N // _BR
    idx = pl.pallas_call(
        _knn_body,
        grid=(B, nb),
        in_specs=[
            pl.BlockSpec((1, _BR, 8), lambda b, i: (b, i, 0)),
            pl.BlockSpec((1, 8, N), lambda b, i: (b, 0, 0)),
        ],
        out_specs=pl.BlockSpec((_BR, _K), lambda b, i: (b * nb + i, 0)),
        out_shape=jax.ShapeDtypeStruct((BN, _K), jnp.int32),
    )(pos_pad, post)

    yk = _make_gather(BN * _K, F)(y, idx.reshape(BN * _K))

    out = pl.pallas_call(
        _mlp2_body,
        grid=(BN // _BR2,),
        in_specs=[
            pl.BlockSpec((_BR2, F), lambda i: (i, 0)),
            pl.BlockSpec((_BR2 * _K, F), lambda i: (i, 0)),
            pl.BlockSpec((F, F), lambda i: (0, 0)),
            pl.BlockSpec((1, F), lambda i: (0, 0)),
        ],
        out_specs=pl.BlockSpec((_BR2, F), lambda i: (i, 0)),
        out_shape=jax.ShapeDtypeStruct((BN, F), jnp.float32),
    )(a, yk, W2, b2.reshape(1, F))

    return out.reshape(B, N, F)


# speedup vs baseline: 37.4379x; 1.3792x over previous
import functools

import jax
import jax.numpy as jnp
from jax import lax
from jax.experimental import pallas as pl
from jax.experimental.pallas import tpu as pltpu
from jax.experimental.pallas import tpu_sc as plsc

_K = 8
_BM = 2048
_BR = 256
_BR2 = 512
_NC = 2
_NS = 16
_NW = _NC * _NS
_CH = 512


def _mlp1_body(x_ref, w1_ref, b1_ref, a_ref, y_ref):
    xb = x_ref[...]
    w = w1_ref[...]
    f = xb.shape[1]
    a_ref[...] = (
        jnp.dot(xb, w[:f], preferred_element_type=jnp.float32) + b1_ref[...]
    )
    y_ref[...] = jnp.dot(xb, w[f:], preferred_element_type=jnp.float32)


def _m11(a, ai, b, bi):
    c = b < a
    return ([jnp.where(c, b, a), jnp.where(c, a, b)],
            [jnp.where(c, bi, ai), jnp.where(c, ai, bi)])


def _m22(A, Ai, B, Bi):
    c0 = B[0] < A[0]
    o0 = jnp.where(c0, B[0], A[0])
    o0i = jnp.where(c0, Bi[0], Ai[0])
    m = jnp.where(c0, A[0], B[0])
    mi = jnp.where(c0, Ai[0], Bi[0])
    c1 = B[1] < A[1]
    o3 = jnp.where(c1, A[1], B[1])
    o3i = jnp.where(c1, Ai[1], Bi[1])
    nv = jnp.where(c1, B[1], A[1])
    ni = jnp.where(c1, Bi[1], Ai[1])
    c2 = nv < m
    return ([o0, jnp.where(c2, nv, m), jnp.where(c2, m, nv), o3],
            [o0i, jnp.where(c2, ni, mi), jnp.where(c2, mi, ni), o3i])


def _cmpx(v, vi, i, j):
    c = v[j] < v[i]
    v[i], v[j] = jnp.where(c, v[j], v[i]), jnp.where(c, v[i], v[j])
    vi[i], vi[j] = jnp.where(c, vi[j], vi[i]), jnp.where(c, vi[i], vi[j])


def _m44_low(A, Ai, B, Bi):
    t, ti = [], []
    for i in range(4):
        c = B[3 - i] < A[i]
        t.append(jnp.where(c, B[3 - i], A[i]))
        ti.append(jnp.where(c, Bi[3 - i], Ai[i]))
    _cmpx(t, ti, 0, 2)
    _cmpx(t, ti, 1, 3)
    _cmpx(t, ti, 0, 1)
    _cmpx(t, ti, 2, 3)
    return t, ti


def _knn_body(pos_ref, post_ref, idx_ref):
    p = pos_ref[0]
    pt = post_ref[0]
    n = pt.shape[1]
    ns = n // 128
    g = jnp.dot(p, pt, preferred_element_type=jnp.float32)
    sqi = jnp.sum(p * p, axis=1, keepdims=True)
    sqj = jnp.sum(pt * pt, axis=0, keepdims=True)
    d = (sqi - 2.0 * g) + sqj
    br = d.shape[0]

    lists = []
    for j in range(0, ns, 2):
        a = d[:, 128 * j:128 * (j + 1)]
        b = d[:, 128 * (j + 1):128 * (j + 2)]
        ja = jnp.full((br, 128), j, jnp.int32)
        jb = jnp.full((br, 128), j + 1, jnp.int32)
        lists.append(_m11(a, ja, b, jb))
    l4 = [_m22(*A, *B) for A, B in zip(lists[0::2], lists[1::2])]
    while len(l4) > 1:
        l4 = [_m44_low(*A, *B) for A, B in zip(l4[0::2], l4[1::2])]
    (s0, s1, s2, s3), (i0, i1, i2, i3) = l4[0]

    lane = lax.broadcasted_iota(jnp.int32, (br, 128), 1)
    big = jnp.int32(1 << 30)
    h, hs1, hs2, hs3 = s0, s1, s2, s3
    ji = [i * 128 + lane for i in (i0, i1, i2, i3)]
    hi, j1, j2, j3 = ji
    cols = []
    inf = jnp.float32(jnp.inf)
    for _ in range(_K):
        m = jnp.min(h, axis=1, keepdims=True)
        lm = jnp.min(jnp.where(h == m, lane, n), axis=1, keepdims=True)
        L = lane == lm
        cols.append(jnp.min(jnp.where(L, hi, big), axis=1))
        h = jnp.where(L, hs1, h)
        hi = jnp.where(L, j1, hi)
        hs1 = jnp.where(L, hs2, hs1)
        j1 = jnp.where(L, j2, j1)
        hs2 = jnp.where(L, hs3, hs2)
        j2 = jnp.where(L, j3, j2)
        hs3 = jnp.where(L, inf, hs3)
    idx = jnp.stack(cols, axis=1)
    idx_ref[...] = idx + pl.program_id(0) * n


def _mlp2_body(a_ref, yk_ref, w2_ref, b2_ref, o_ref):
    a = a_ref[...]
    br2, f = a.shape
    yk = yk_ref[...].reshape(br2, _K, f)
    h = jnp.maximum(a[:, None, :] + yk, 0.0)
    o = jnp.dot(
        h.reshape(br2 * _K, f), w2_ref[...], preferred_element_type=jnp.float32
    )
    o_ref[...] = jnp.max(o.reshape(br2, _K, f), axis=1) + b2_ref[...]


@functools.cache
def _make_gather(rows, d):
    per_w = rows // _NW
    n_ch = per_w // _CH
    mesh = plsc.VectorSubcoreMesh(core_axis_name="c", subcore_axis_name="s")

    @functools.partial(
        pl.kernel,
        out_type=jax.ShapeDtypeStruct((rows, d), jnp.float32),
        mesh=mesh,
        scratch_types=[
            pltpu.VMEM((_CH,), jnp.int32),
            pltpu.VMEM((_CH, d), jnp.float32),
            pltpu.SemaphoreType.DMA,
        ],
    )
    def gather(table_hbm, idx_hbm, out_hbm, idx_v, rows_v, sem):
        wid = lax.axis_index("s") * _NC + lax.axis_index("c")
        base = wid * per_w

        def body(i, carry):
            off = base + i * _CH
            pltpu.sync_copy(idx_hbm.at[pl.ds(off, _CH)], idx_v)
            pltpu.async_copy(table_hbm.at[idx_v], rows_v, sem).wait()
            pltpu.sync_copy(rows_v, out_hbm.at[pl.ds(off, _CH)])
            return carry

        lax.fori_loop(0, n_ch, body, 0)

    return gather


def kernel(x, pos, W1, b1, W2, b2):
    B, N, F = x.shape
    BN = B * N
    xf = x.reshape(BN, F)

    a, y = pl.pallas_call(
        _mlp1_body,
        grid=(BN // _BM,),
        in_specs=[
            pl.BlockSpec((_BM, F), lambda i: (i, 0)),
            pl.BlockSpec((2 * F, F), lambda i: (0, 0)),
            pl.BlockSpec((1, F), lambda i: (0, 0)),
        ],
        out_specs=[
            pl.BlockSpec((_BM, F), lambda i: (i, 0)),
            pl.BlockSpec((_BM, F), lambda i: (i, 0)),
        ],
        out_shape=[
            jax.ShapeDtypeStruct((BN, F), jnp.float32),
            jax.ShapeDtypeStruct((BN, F), jnp.float32),
        ],
    )(xf, W1, b1.reshape(1, F))

    pos_pad = jnp.pad(pos, ((0, 0), (0, 0), (0, 5)))
    post = jnp.transpose(pos_pad, (0, 2, 1))
    nb = N // _BR
    idx = pl.pallas_call(
        _knn_body,
        grid=(B, nb),
        in_specs=[
            pl.BlockSpec((1, _BR, 8), lambda b, i: (b, i, 0)),
            pl.BlockSpec((1, 8, N), lambda b, i: (b, 0, 0)),
        ],
        out_specs=pl.BlockSpec((_BR, _K), lambda b, i: (b * nb + i, 0)),
        out_shape=jax.ShapeDtypeStruct((BN, _K), jnp.int32),
    )(pos_pad, post)

    yk = _make_gather(BN * _K, F)(y, idx.reshape(BN * _K))

    out = pl.pallas_call(
        _mlp2_body,
        grid=(BN // _BR2,),
        in_specs=[
            pl.BlockSpec((_BR2, F), lambda i: (i, 0)),
            pl.BlockSpec((_BR2 * _K, F), lambda i: (i, 0)),
            pl.BlockSpec((F, F), lambda i: (0, 0)),
            pl.BlockSpec((1, F), lambda i: (0, 0)),
        ],
        out_specs=pl.BlockSpec((_BR2, F), lambda i: (i, 0)),
        out_shape=jax.ShapeDtypeStruct((BN, F), jnp.float32),
    )(a, yk, W2, b2.reshape(1, F))

    return out.reshape(B, N, F)
